# baseline (speedup 1.0000x reference)
import functools

import jax
import jax.numpy as jnp
from jax import lax
from jax.experimental import pallas as pl
from jax.experimental.pallas import tpu as pltpu
from jax.experimental.pallas import tpu_sc as plsc

BATCH = 16384
D = 32
GROUP = 128 // D
NROW = 100000 // GROUP
NC = 2
NS = 16
NW = NC * NS
B_PER_W = BATCH // NW


def _sc_gather4(u_mlp_tab, i_mlp_tab, u_mf_tab, i_mf_tab, uidx, iidx):
  mesh = plsc.VectorSubcoreMesh(core_axis_name="c", subcore_axis_name="s")
  row = jax.ShapeDtypeStruct((BATCH, 128), jnp.float32)

  ch = 64
  nch = B_PER_W // ch
  buf = pltpu.VMEM((ch, 128), jnp.float32)

  @functools.partial(
      pl.kernel,
      mesh=mesh,
      out_type=[row, row, row, row],
      compiler_params=pltpu.CompilerParams(use_tc_tiling_on_sc=True),
      scratch_types=[
          pltpu.VMEM((B_PER_W,), jnp.int32),
          pltpu.VMEM((B_PER_W,), jnp.int32),
          buf, buf, buf, buf,
          buf, buf, buf, buf,
          pltpu.SemaphoreType.DMA,
          pltpu.SemaphoreType.DMA,
          pltpu.SemaphoreType.DMA,
          pltpu.SemaphoreType.DMA,
      ],
  )
  def k(ut_hbm, it_hbm, umf_hbm, imf_hbm, ui_hbm, ii_hbm,
        o1, o2, o3, o4, ui_v, ii_v,
        a1, a2, a3, a4, b1_, b2_, b3_, b4_, gsem0, gsem1, osem0, osem1):
    wid = lax.axis_index("s") * NC + lax.axis_index("c")
    base = wid * B_PER_W
    pltpu.sync_copy(ui_hbm.at[pl.ds(base, B_PER_W)], ui_v)
    pltpu.sync_copy(ii_hbm.at[pl.ds(base, B_PER_W)], ii_v)
    bufs = [(a1, a2, a3, a4), (b1_, b2_, b3_, b4_)]
    gsems = [gsem0, gsem1]
    osems = [osem0, osem1]
    tabs = (ut_hbm, it_hbm, umf_hbm, imf_hbm)
    outs = (o1, o2, o3, o4)

    def fire_gathers(c):
      s = c % 2
      off = c * ch
      idxs = (ui_v, ii_v, ui_v, ii_v)
      return [
          pltpu.async_copy(tabs[t].at[idxs[t].at[pl.ds(off, ch)]],
                           bufs[s][t], gsems[s])
          for t in range(4)
      ]

    def fire_writes(c):
      s = c % 2
      off = base + c * ch
      return [
          pltpu.async_copy(bufs[s][t], outs[t].at[pl.ds(off, ch)], osems[s])
          for t in range(4)
      ]

    pend_g = {0: fire_gathers(0)}
    pend_w = {}
    for c in range(nch):
      if c + 1 < nch:
        if c - 1 >= 0:
          for w in pend_w.pop(c - 1):
            w.wait()
        pend_g[c + 1] = fire_gathers(c + 1)
      for g in pend_g.pop(c):
        g.wait()
      pend_w[c] = fire_writes(c)
    for c in list(pend_w):
      for w in pend_w.pop(c):
        w.wait()

  return k(u_mlp_tab, i_mlp_tab, u_mf_tab, i_mf_tab, uidx, iidx)


def _tc_mlp(gu_mlp, gi_mlp, gu_mf, gi_mf, uq, iq, W1a, W1b, b1, W2t, b2,
            W3t, b3, wo_mlp, wo_mf, bo):
  blk = 4096
  grid = (BATCH // blk,)

  def body(u_ref, i_ref, umf_ref, imf_ref, uq_ref, iq_ref,
           w1a_ref, w1b_ref, b1_ref, w2_ref, b2_ref, w3_ref, b3_ref,
           womlp_ref, womf_ref, bo_ref, o_ref):
    col_q = lax.broadcasted_iota(jnp.int32, (1, 128), 1) // D

    def sel(ref, q_ref):
      m = jnp.where(col_q == q_ref[...], ref[...], 0.0)
      return m[:, :D] + m[:, D:2 * D] + m[:, 2 * D:3 * D] + m[:, 3 * D:]

    u = sel(u_ref, uq_ref)
    it = sel(i_ref, iq_ref)
    h = jnp.dot(u, w1a_ref[...], preferred_element_type=jnp.float32)
    h += jnp.dot(it, w1b_ref[...], preferred_element_type=jnp.float32)
    h = jnp.maximum(h + b1_ref[...], 0.0)
    h = jnp.dot(h, w2_ref[...], preferred_element_type=jnp.float32)
    h = jnp.maximum(h + b2_ref[...], 0.0)
    h = jnp.dot(h, w3_ref[...], preferred_element_type=jnp.float32)
    h = jnp.maximum(h + b3_ref[...], 0.0)
    mf = sel(umf_ref, uq_ref) * sel(imf_ref, iq_ref)
    logit = jnp.dot(h, womlp_ref[...], preferred_element_type=jnp.float32)
    logit += jnp.dot(mf, womf_ref[...], preferred_element_type=jnp.float32)
    o_ref[...] = jax.nn.sigmoid(logit + bo_ref[...])

  rows = pl.BlockSpec((blk, 128), lambda i: (i, 0))
  qcol = pl.BlockSpec((blk, 1), lambda i: (i, 0))
  full = lambda s: pl.BlockSpec(s, lambda i: tuple(0 for _ in s))
  return pl.pallas_call(
      body,
      grid=grid,
      in_specs=[
          rows, rows, rows, rows, qcol, qcol,
          full((D, D)), full((D, D)), full((1, D)),
          full((D, 16)), full((1, 16)),
          full((16, 8)), full((1, 8)),
          full((8, 1)), full((D, 1)), full((1, 1)),
      ],
      out_specs=pl.BlockSpec((blk, 1), lambda i: (i, 0)),
      out_shape=jax.ShapeDtypeStruct((BATCH, 1), jnp.float32),
  )(gu_mlp, gi_mlp, gu_mf, gi_mf, uq, iq, W1a, W1b, b1, W2t, b2, W3t, b3,
    wo_mlp, wo_mf, bo)


def kernel(user_indices, item_indices, emb_user_mlp, emb_item_mlp,
           emb_user_mf, emb_item_mf, W1, b1, W2, b2, W3, b3, Wo, bo):
  uidx = user_indices.astype(jnp.int32)
  iidx = item_indices.astype(jnp.int32)
  ug = uidx // GROUP
  ig = iidx // GROUP
  uq = (uidx % GROUP).reshape(-1, 1)
  iq = (iidx % GROUP).reshape(-1, 1)

  wide = lambda t: t.reshape(NROW, 128)
  gu_mlp, gi_mlp, gu_mf, gi_mf = _sc_gather4(
      wide(emb_user_mlp), wide(emb_item_mlp),
      wide(emb_user_mf), wide(emb_item_mf), ug, ig)

  W1a = W1[:, :D].T
  W1b = W1[:, D:].T
  W2t = W2.T
  W3t = W3.T
  wo_mlp = Wo[:, :8].T
  wo_mf = Wo[:, 8:].T

  out = _tc_mlp(gu_mlp, gi_mlp, gu_mf, gi_mf, uq, iq,
                W1a, W1b, b1.reshape(1, -1), W2t, b2.reshape(1, -1),
                W3t, b3.reshape(1, -1), wo_mlp, wo_mf, bo.reshape(1, 1))
  return out.reshape(BATCH)
